# Initial kernel scaffold; baseline (speedup 1.0000x reference)
import jax
import jax.numpy as jnp
from jax.experimental import pallas as pl


def kernel(x, edge_index, W1, b1, W2, b2):
    raise NotImplementedError("write your pallas kernel here")


# speedup vs baseline: 8.0413x; 8.0413x over previous
import functools

import jax
import jax.numpy as jnp
from jax import lax
from jax.experimental import pallas as pl
from jax.experimental.pallas import tpu as pltpu
from jax.experimental.pallas import tpu_sc as plsc

N = 10000
E = 320000
NACC = 10240
WIN = 128
NTILE = 16
NWMAIN = 160
CHW = 8
NWHIST = 80
RB = 2000


def _sc_mesh():
    return plsc.VectorSubcoreMesh(core_axis_name="c", subcore_axis_name="s")


def _degree_hist(col_h, ones_w, zeros_row):
    @functools.partial(
        pl.kernel,
        out_type=jax.ShapeDtypeStruct((2, NACC), jnp.float32),
        mesh=_sc_mesh(),
        scratch_types=[
            pltpu.VMEM((NWHIST, WIN), jnp.int32),
            pltpu.VMEM((WIN,), jnp.float32),
            pltpu.VMEM_SHARED((NACC,), jnp.float32),
        ],
    )
    def hist(col_hbm, ones_hbm, zeros_hbm, out_hbm, cidx_v, ones_v, deg_sh):
        c = lax.axis_index("c")
        s = lax.axis_index("s")
        w = s * 2 + c
        pltpu.sync_copy(col_hbm.at[w], cidx_v)
        pltpu.sync_copy(ones_hbm, ones_v)
        pltpu.sync_copy(zeros_hbm, deg_sh.at[pl.ds(s * (NACC // NTILE), NACC // NTILE)])
        plsc.subcore_barrier()

        def body(j, carry):
            pltpu.sync_copy(ones_v, deg_sh.at[cidx_v.at[j]], add=True)
            return carry

        lax.fori_loop(0, NWHIST, body, 0)
        plsc.subcore_barrier()
        sl = pl.ds(s * (NACC // NTILE), NACC // NTILE)
        pltpu.sync_copy(deg_sh.at[sl], out_hbm.at[c, sl])

    return hist(col_h, ones_w, zeros_row)


def _spmm(g0, g1, rows3, cols3, zeros_acc, F):
    @functools.partial(
        pl.kernel,
        out_type=(
            jax.ShapeDtypeStruct((NACC, F), jnp.float32),
            jax.ShapeDtypeStruct((NACC, F), jnp.float32),
        ),
        mesh=_sc_mesh(),
        scratch_types=[
            pltpu.VMEM((CHW, WIN), jnp.int32),
            pltpu.VMEM((CHW, WIN), jnp.int32),
            pltpu.VMEM((WIN, F), jnp.float32),
            pltpu.VMEM_SHARED((NACC, F), jnp.float32),
            pltpu.SemaphoreType.DMA,
        ],
    )
    def spmm(g0_hbm, g1_hbm, rows_hbm, cols_hbm, zeros_hbm,
             out0_hbm, out1_hbm, ridx_v, cidx_v, rows_v, acc_sh, sem):
        c = lax.axis_index("c")
        s = lax.axis_index("s")
        pltpu.sync_copy(zeros_hbm, acc_sh.at[pl.ds(s * (NACC // NTILE), NACC // NTILE)])
        plsc.subcore_barrier()

        def run(g_hbm):
            def outer(t, carry):
                pltpu.sync_copy(rows_hbm.at[s, pl.ds(t * CHW, CHW)], ridx_v)
                pltpu.sync_copy(cols_hbm.at[s, pl.ds(t * CHW, CHW)], cidx_v)

                def body(j, carry2):
                    pltpu.async_copy(g_hbm.at[ridx_v.at[j]], rows_v, sem).wait()
                    pltpu.sync_copy(rows_v, acc_sh.at[cidx_v.at[j]], add=True)
                    return carry2

                return lax.fori_loop(0, CHW, body, carry)

            lax.fori_loop(0, NWMAIN // CHW, outer, 0)

        @pl.when(c == 0)
        def _():
            run(g0_hbm)

        @pl.when(c == 1)
        def _():
            run(g1_hbm)

        plsc.subcore_barrier()
        osl = pl.ds(s * (NACC // NTILE), NACC // NTILE)

        @pl.when(c == 0)
        def _():
            pltpu.sync_copy(acc_sh.at[osl], out0_hbm.at[osl])

        @pl.when(c == 1)
        def _():
            pltpu.sync_copy(acc_sh.at[osl], out1_hbm.at[osl])

    return spmm(g0, g1, rows3, cols3, zeros_acc)


def _spmm_es(g, rows_w, cols_w, zeros_acc):
    @functools.partial(
        pl.kernel,
        out_type=(
            jax.ShapeDtypeStruct((NACC, 128), jnp.float32),
            jax.ShapeDtypeStruct((NACC, 128), jnp.float32),
        ),
        mesh=_sc_mesh(),
        scratch_types=[
            pltpu.VMEM((CHW, WIN), jnp.int32),
            pltpu.VMEM((CHW, WIN), jnp.int32),
            pltpu.VMEM((WIN, 128), jnp.float32),
            pltpu.VMEM_SHARED((NACC, 128), jnp.float32),
            pltpu.SemaphoreType.DMA,
        ],
    )
    def spmm(g_hbm, rows_hbm, cols_hbm, zeros_hbm,
             out0_hbm, out1_hbm, ridx_v, cidx_v, rows_v, acc_sh, sem):
        c = lax.axis_index("c")
        s = lax.axis_index("s")
        w = s * 2 + c
        pltpu.sync_copy(zeros_hbm, acc_sh.at[pl.ds(s * (NACC // NTILE), NACC // NTILE)])
        plsc.subcore_barrier()

        def outer(t, carry):
            pltpu.sync_copy(rows_hbm.at[w, pl.ds(t * CHW, CHW)], ridx_v)
            pltpu.sync_copy(cols_hbm.at[w, pl.ds(t * CHW, CHW)], cidx_v)

            def body(j, carry2):
                pltpu.async_copy(g_hbm.at[ridx_v.at[j]], rows_v, sem).wait()
                pltpu.sync_copy(rows_v, acc_sh.at[cidx_v.at[j]], add=True)
                return carry2

            return lax.fori_loop(0, CHW, body, carry)

        lax.fori_loop(0, NWHIST // CHW, outer, 0)

        plsc.subcore_barrier()
        osl = pl.ds(s * (NACC // NTILE), NACC // NTILE)

        @pl.when(c == 0)
        def _():
            pltpu.sync_copy(acc_sh.at[osl], out0_hbm.at[osl])

        @pl.when(c == 1)
        def _():
            pltpu.sync_copy(acc_sh.at[osl], out1_hbm.at[osl])

    return spmm(g, rows_w, cols_w, zeros_acc)


def _tc1(x, W1, d0, d1):
    def body(x_r, w_r, d0_r, d1_r, g1a_r, g1b_r, dis_r):
        deg = d0_r[...] + d1_r[...] + 1.0
        dis = lax.rsqrt(deg)
        g = dis * jnp.dot(x_r[...], w_r[...], preferred_element_type=jnp.float32)
        g1a_r[...] = g[:, :128]
        g1b_r[...] = g[:, 128:]
        dis_r[...] = dis

    return pl.pallas_call(
        body,
        grid=(N // RB,),
        in_specs=[
            pl.BlockSpec((RB, 128), lambda i: (i, 0)),
            pl.BlockSpec((128, 256), lambda i: (0, 0)),
            pl.BlockSpec((RB, 1), lambda i: (i, 0)),
            pl.BlockSpec((RB, 1), lambda i: (i, 0)),
        ],
        out_specs=[
            pl.BlockSpec((RB, 128), lambda i: (i, 0)),
            pl.BlockSpec((RB, 128), lambda i: (i, 0)),
            pl.BlockSpec((RB, 1), lambda i: (i, 0)),
        ],
        out_shape=[
            jax.ShapeDtypeStruct((N, 128), jnp.float32),
            jax.ShapeDtypeStruct((N, 128), jnp.float32),
            jax.ShapeDtypeStruct((N, 1), jnp.float32),
        ],
    )(x, W1, d0, d1)


def _tc2(s1a, s1b, g1a, g1b, dis, b1, W2):
    def body(s1a_r, s1b_r, g1a_r, g1b_r, dis_r, b1_r, w2_r, g2_r):
        d = dis_r[...]
        pre0 = d * (s1a_r[...] + g1a_r[...]) + b1_r[:, :128]
        pre1 = d * (s1b_r[...] + g1b_r[...]) + b1_r[:, 128:]
        h2 = jnp.concatenate([jnp.maximum(pre0, 0.0), jnp.maximum(pre1, 0.0)], axis=1)
        g2_r[...] = d * jnp.dot(h2, w2_r[...], preferred_element_type=jnp.float32)

    return pl.pallas_call(
        body,
        grid=(N // RB,),
        in_specs=[
            pl.BlockSpec((RB, 128), lambda i: (i, 0)),
            pl.BlockSpec((RB, 128), lambda i: (i, 0)),
            pl.BlockSpec((RB, 128), lambda i: (i, 0)),
            pl.BlockSpec((RB, 128), lambda i: (i, 0)),
            pl.BlockSpec((RB, 1), lambda i: (i, 0)),
            pl.BlockSpec((1, 256), lambda i: (0, 0)),
            pl.BlockSpec((256, 128), lambda i: (0, 0)),
        ],
        out_specs=pl.BlockSpec((RB, 128), lambda i: (i, 0)),
        out_shape=jax.ShapeDtypeStruct((N, 128), jnp.float32),
    )(s1a, s1b, g1a, g1b, dis, b1, W2)


def _tc3(s2p0, s2p1, g2, dis, b2):
    def body(p0_r, p1_r, g2_r, dis_r, b2_r, out_r):
        d = dis_r[...]
        out_r[...] = d * (p0_r[...] + p1_r[...] + g2_r[...]) + b2_r[...]

    return pl.pallas_call(
        body,
        grid=(N // RB,),
        in_specs=[
            pl.BlockSpec((RB, 128), lambda i: (i, 0)),
            pl.BlockSpec((RB, 128), lambda i: (i, 0)),
            pl.BlockSpec((RB, 128), lambda i: (i, 0)),
            pl.BlockSpec((RB, 1), lambda i: (i, 0)),
            pl.BlockSpec((1, 128), lambda i: (0, 0)),
        ],
        out_specs=pl.BlockSpec((RB, 128), lambda i: (i, 0)),
        out_shape=jax.ShapeDtypeStruct((N, 128), jnp.float32),
    )(s2p0, s2p1, g2, dis, b2)


def kernel(x, edge_index, W1, b1, W2, b2):
    row = edge_index[0].astype(jnp.int32)
    col = edge_index[1].astype(jnp.int32)

    pad_m = NTILE * NWMAIN * WIN - E
    junk_m = N + (jnp.arange(pad_m, dtype=jnp.int32) % (NACC - N))
    rows3 = jnp.concatenate([row, jnp.zeros((pad_m,), jnp.int32)]).reshape(NTILE, NWMAIN, WIN)
    cols3 = jnp.concatenate([col, junk_m]).reshape(NTILE, NWMAIN, WIN)

    pad_w = 32 * NWHIST * WIN - E
    junk_w = N + (jnp.arange(pad_w, dtype=jnp.int32) % (NACC - N))
    rows_w = jnp.concatenate([row, jnp.zeros((pad_w,), jnp.int32)]).reshape(32, NWHIST, WIN)
    cols_w = jnp.concatenate([col, junk_w]).reshape(32, NWHIST, WIN)

    ones_w = jnp.ones((WIN,), jnp.float32)
    zeros_row = jnp.zeros((NACC // NTILE,), jnp.float32)
    zeros_acc128 = jnp.zeros((NACC // NTILE, 128), jnp.float32)

    degp = _degree_hist(cols_w, ones_w, zeros_row)
    d0 = degp[0, :N].reshape(N, 1)
    d1 = degp[1, :N].reshape(N, 1)

    g1a, g1b, dis = _tc1(x, W1, d0, d1)
    s1a, s1b = _spmm(g1a, g1b, rows3, cols3, zeros_acc128, 128)
    g2 = _tc2(s1a, s1b, g1a, g1b, dis, b1.reshape(1, 256), W2)
    s2p0, s2p1 = _spmm_es(g2, rows_w, cols_w, zeros_acc128)
    return _tc3(s2p0, s2p1, g2, dis, b2.reshape(1, 128))


# speedup vs baseline: 9.3164x; 1.1586x over previous
import functools

import jax
import jax.numpy as jnp
from jax import lax
from jax.experimental import pallas as pl
from jax.experimental.pallas import tpu as pltpu
from jax.experimental.pallas import tpu_sc as plsc

N = 10000
E = 320000
NACC = 10240
WIN = 128
NTILE = 16
NWMAIN = 160
CHW = 16
NWHIST = 80
RB = 2000


def _sc_mesh():
    return plsc.VectorSubcoreMesh(core_axis_name="c", subcore_axis_name="s")


def _degree_hist(col_h, ones_w, zeros_row):
    @functools.partial(
        pl.kernel,
        out_type=jax.ShapeDtypeStruct((2, NACC), jnp.float32),
        mesh=_sc_mesh(),
        scratch_types=[
            pltpu.VMEM((NWHIST, WIN), jnp.int32),
            pltpu.VMEM((WIN,), jnp.float32),
            pltpu.VMEM_SHARED((NACC,), jnp.float32),
        ],
    )
    def hist(col_hbm, ones_hbm, zeros_hbm, out_hbm, cidx_v, ones_v, deg_sh):
        c = lax.axis_index("c")
        s = lax.axis_index("s")
        w = s * 2 + c
        pltpu.sync_copy(col_hbm.at[w], cidx_v)
        pltpu.sync_copy(ones_hbm, ones_v)
        pltpu.sync_copy(zeros_hbm, deg_sh.at[pl.ds(s * (NACC // NTILE), NACC // NTILE)])
        plsc.subcore_barrier()

        def body(j, carry):
            pltpu.sync_copy(ones_v, deg_sh.at[cidx_v.at[j]], add=True)
            return carry

        lax.fori_loop(0, NWHIST, body, 0)
        plsc.subcore_barrier()
        sl = pl.ds(s * (NACC // NTILE), NACC // NTILE)
        pltpu.sync_copy(deg_sh.at[sl], out_hbm.at[c, sl])

    return hist(col_h, ones_w, zeros_row)


def _edge_pipeline(g_hbm, rows_hbm, cols_hbm, widx, nchunks,
                   ridx_v, cidx_v, bufs, gsems, ssems, acc_sh):
    def chunk(t, carry):
        pltpu.sync_copy(rows_hbm.at[widx, pl.ds(t * CHW, CHW)], ridx_v)
        pltpu.sync_copy(cols_hbm.at[widx, pl.ds(t * CHW, CHW)], cidx_v)
        descs_g = [None, None]
        descs_s = [None, None]
        descs_g[0] = pltpu.async_copy(g_hbm.at[ridx_v.at[0]], bufs[0], gsems[0])
        for j in range(CHW):
            b = j & 1
            nb = 1 - b
            if j >= 1:
                descs_s[nb].wait()
            if j + 1 < CHW:
                descs_g[nb] = pltpu.async_copy(
                    g_hbm.at[ridx_v.at[j + 1]], bufs[nb], gsems[nb])
            descs_g[b].wait()
            descs_s[b] = pltpu.async_copy(
                bufs[b], acc_sh.at[cidx_v.at[j]], ssems[b], add=True)
        descs_s[(CHW - 1) & 1].wait()
        return carry

    lax.fori_loop(0, nchunks, chunk, 0)


def _spmm(g0, g1, rows3, cols3, zeros_acc, F):
    @functools.partial(
        pl.kernel,
        out_type=(
            jax.ShapeDtypeStruct((NACC, F), jnp.float32),
            jax.ShapeDtypeStruct((NACC, F), jnp.float32),
        ),
        mesh=_sc_mesh(),
        scratch_types=[
            pltpu.VMEM((CHW, WIN), jnp.int32),
            pltpu.VMEM((CHW, WIN), jnp.int32),
            pltpu.VMEM((WIN, F), jnp.float32),
            pltpu.VMEM((WIN, F), jnp.float32),
            pltpu.VMEM_SHARED((NACC, F), jnp.float32),
            pltpu.SemaphoreType.DMA,
            pltpu.SemaphoreType.DMA,
            pltpu.SemaphoreType.DMA,
            pltpu.SemaphoreType.DMA,
        ],
    )
    def spmm(g0_hbm, g1_hbm, rows_hbm, cols_hbm, zeros_hbm,
             out0_hbm, out1_hbm, ridx_v, cidx_v, buf0, buf1, acc_sh,
             semg0, semg1, sems0, sems1):
        c = lax.axis_index("c")
        s = lax.axis_index("s")
        pltpu.sync_copy(zeros_hbm, acc_sh.at[pl.ds(s * (NACC // NTILE), NACC // NTILE)])
        plsc.subcore_barrier()

        @pl.when(c == 0)
        def _():
            _edge_pipeline(g0_hbm, rows_hbm, cols_hbm, s, NWMAIN // CHW,
                           ridx_v, cidx_v, (buf0, buf1), (semg0, semg1),
                           (sems0, sems1), acc_sh)

        @pl.when(c == 1)
        def _():
            _edge_pipeline(g1_hbm, rows_hbm, cols_hbm, s, NWMAIN // CHW,
                           ridx_v, cidx_v, (buf0, buf1), (semg0, semg1),
                           (sems0, sems1), acc_sh)

        plsc.subcore_barrier()
        osl = pl.ds(s * (NACC // NTILE), NACC // NTILE)

        @pl.when(c == 0)
        def _():
            pltpu.sync_copy(acc_sh.at[osl], out0_hbm.at[osl])

        @pl.when(c == 1)
        def _():
            pltpu.sync_copy(acc_sh.at[osl], out1_hbm.at[osl])

    return spmm(g0, g1, rows3, cols3, zeros_acc)


def _spmm_es(g, rows_w, cols_w, zeros_acc):
    @functools.partial(
        pl.kernel,
        out_type=(
            jax.ShapeDtypeStruct((NACC, 128), jnp.float32),
            jax.ShapeDtypeStruct((NACC, 128), jnp.float32),
        ),
        mesh=_sc_mesh(),
        scratch_types=[
            pltpu.VMEM((CHW, WIN), jnp.int32),
            pltpu.VMEM((CHW, WIN), jnp.int32),
            pltpu.VMEM((WIN, 128), jnp.float32),
            pltpu.VMEM((WIN, 128), jnp.float32),
            pltpu.VMEM_SHARED((NACC, 128), jnp.float32),
            pltpu.SemaphoreType.DMA,
            pltpu.SemaphoreType.DMA,
            pltpu.SemaphoreType.DMA,
            pltpu.SemaphoreType.DMA,
        ],
    )
    def spmm(g_hbm, rows_hbm, cols_hbm, zeros_hbm,
             out0_hbm, out1_hbm, ridx_v, cidx_v, buf0, buf1, acc_sh,
             semg0, semg1, sems0, sems1):
        c = lax.axis_index("c")
        s = lax.axis_index("s")
        w = s * 2 + c
        pltpu.sync_copy(zeros_hbm, acc_sh.at[pl.ds(s * (NACC // NTILE), NACC // NTILE)])
        plsc.subcore_barrier()

        _edge_pipeline(g_hbm, rows_hbm, cols_hbm, w, NWHIST // CHW,
                       ridx_v, cidx_v, (buf0, buf1), (semg0, semg1),
                       (sems0, sems1), acc_sh)

        plsc.subcore_barrier()
        osl = pl.ds(s * (NACC // NTILE), NACC // NTILE)

        @pl.when(c == 0)
        def _():
            pltpu.sync_copy(acc_sh.at[osl], out0_hbm.at[osl])

        @pl.when(c == 1)
        def _():
            pltpu.sync_copy(acc_sh.at[osl], out1_hbm.at[osl])

    return spmm(g, rows_w, cols_w, zeros_acc)


def _tc1(x, W1, d0, d1):
    def body(x_r, w_r, d0_r, d1_r, g1a_r, g1b_r, dis_r):
        deg = d0_r[...] + d1_r[...] + 1.0
        dis = lax.rsqrt(deg)
        g = dis * jnp.dot(x_r[...], w_r[...], preferred_element_type=jnp.float32)
        g1a_r[...] = g[:, :128]
        g1b_r[...] = g[:, 128:]
        dis_r[...] = dis

    return pl.pallas_call(
        body,
        grid=(N // RB,),
        in_specs=[
            pl.BlockSpec((RB, 128), lambda i: (i, 0)),
            pl.BlockSpec((128, 256), lambda i: (0, 0)),
            pl.BlockSpec((RB, 1), lambda i: (i, 0)),
            pl.BlockSpec((RB, 1), lambda i: (i, 0)),
        ],
        out_specs=[
            pl.BlockSpec((RB, 128), lambda i: (i, 0)),
            pl.BlockSpec((RB, 128), lambda i: (i, 0)),
            pl.BlockSpec((RB, 1), lambda i: (i, 0)),
        ],
        out_shape=[
            jax.ShapeDtypeStruct((N, 128), jnp.float32),
            jax.ShapeDtypeStruct((N, 128), jnp.float32),
            jax.ShapeDtypeStruct((N, 1), jnp.float32),
        ],
    )(x, W1, d0, d1)


def _tc2(s1a, s1b, g1a, g1b, dis, b1, W2):
    def body(s1a_r, s1b_r, g1a_r, g1b_r, dis_r, b1_r, w2_r, g2_r):
        d = dis_r[...]
        pre0 = d * (s1a_r[...] + g1a_r[...]) + b1_r[:, :128]
        pre1 = d * (s1b_r[...] + g1b_r[...]) + b1_r[:, 128:]
        h2 = jnp.concatenate([jnp.maximum(pre0, 0.0), jnp.maximum(pre1, 0.0)], axis=1)
        g2_r[...] = d * jnp.dot(h2, w2_r[...], preferred_element_type=jnp.float32)

    return pl.pallas_call(
        body,
        grid=(N // RB,),
        in_specs=[
            pl.BlockSpec((RB, 128), lambda i: (i, 0)),
            pl.BlockSpec((RB, 128), lambda i: (i, 0)),
            pl.BlockSpec((RB, 128), lambda i: (i, 0)),
            pl.BlockSpec((RB, 128), lambda i: (i, 0)),
            pl.BlockSpec((RB, 1), lambda i: (i, 0)),
            pl.BlockSpec((1, 256), lambda i: (0, 0)),
            pl.BlockSpec((256, 128), lambda i: (0, 0)),
        ],
        out_specs=pl.BlockSpec((RB, 128), lambda i: (i, 0)),
        out_shape=jax.ShapeDtypeStruct((N, 128), jnp.float32),
    )(s1a, s1b, g1a, g1b, dis, b1, W2)


def _tc3(s2p0, s2p1, g2, dis, b2):
    def body(p0_r, p1_r, g2_r, dis_r, b2_r, out_r):
        d = dis_r[...]
        out_r[...] = d * (p0_r[...] + p1_r[...] + g2_r[...]) + b2_r[...]

    return pl.pallas_call(
        body,
        grid=(N // RB,),
        in_specs=[
            pl.BlockSpec((RB, 128), lambda i: (i, 0)),
            pl.BlockSpec((RB, 128), lambda i: (i, 0)),
            pl.BlockSpec((RB, 128), lambda i: (i, 0)),
            pl.BlockSpec((RB, 1), lambda i: (i, 0)),
            pl.BlockSpec((1, 128), lambda i: (0, 0)),
        ],
        out_specs=pl.BlockSpec((RB, 128), lambda i: (i, 0)),
        out_shape=jax.ShapeDtypeStruct((N, 128), jnp.float32),
    )(s2p0, s2p1, g2, dis, b2)


def kernel(x, edge_index, W1, b1, W2, b2):
    row = edge_index[0].astype(jnp.int32)
    col = edge_index[1].astype(jnp.int32)

    pad_m = NTILE * NWMAIN * WIN - E
    junk_m = N + (jnp.arange(pad_m, dtype=jnp.int32) % (NACC - N))
    rows3 = jnp.concatenate([row, jnp.zeros((pad_m,), jnp.int32)]).reshape(NTILE, NWMAIN, WIN)
    cols3 = jnp.concatenate([col, junk_m]).reshape(NTILE, NWMAIN, WIN)

    pad_w = 32 * NWHIST * WIN - E
    junk_w = N + (jnp.arange(pad_w, dtype=jnp.int32) % (NACC - N))
    rows_w = jnp.concatenate([row, jnp.zeros((pad_w,), jnp.int32)]).reshape(32, NWHIST, WIN)
    cols_w = jnp.concatenate([col, junk_w]).reshape(32, NWHIST, WIN)

    ones_w = jnp.ones((WIN,), jnp.float32)
    zeros_row = jnp.zeros((NACC // NTILE,), jnp.float32)
    zeros_acc128 = jnp.zeros((NACC // NTILE, 128), jnp.float32)

    degp = _degree_hist(cols_w, ones_w, zeros_row)
    d0 = degp[0, :N].reshape(N, 1)
    d1 = degp[1, :N].reshape(N, 1)

    g1a, g1b, dis = _tc1(x, W1, d0, d1)
    s1a, s1b = _spmm(g1a, g1b, rows3, cols3, zeros_acc128, 128)
    g2 = _tc2(s1a, s1b, g1a, g1b, dis, b1.reshape(1, 256), W2)
    s2p0, s2p1 = _spmm_es(g2, rows_w, cols_w, zeros_acc128)
    return _tc3(s2p0, s2p1, g2, dis, b2.reshape(1, 128))


# speedup vs baseline: 9.6002x; 1.0305x over previous
import functools

import jax
import jax.numpy as jnp
from jax import lax
from jax.experimental import pallas as pl
from jax.experimental.pallas import tpu as pltpu
from jax.experimental.pallas import tpu_sc as plsc

N = 10000
E = 320000
NACC = 10240
WIN = 128
NTILE = 16
NWMAIN = 160
CHW = 16
NWHIST = 80
RB = 2000


def _sc_mesh():
    return plsc.VectorSubcoreMesh(core_axis_name="c", subcore_axis_name="s")


def _degree_hist(col_h, ones_w, zeros_row):
    @functools.partial(
        pl.kernel,
        out_type=jax.ShapeDtypeStruct((2, NACC), jnp.float32),
        mesh=_sc_mesh(),
        scratch_types=[
            pltpu.VMEM((NWHIST, WIN), jnp.int32),
            pltpu.VMEM((WIN,), jnp.float32),
            pltpu.VMEM_SHARED((NACC,), jnp.float32),
        ],
    )
    def hist(col_hbm, ones_hbm, zeros_hbm, out_hbm, cidx_v, ones_v, deg_sh):
        c = lax.axis_index("c")
        s = lax.axis_index("s")
        w = s * 2 + c
        pltpu.sync_copy(col_hbm.at[w], cidx_v)
        pltpu.sync_copy(ones_hbm, ones_v)
        pltpu.sync_copy(zeros_hbm, deg_sh.at[pl.ds(s * (NACC // NTILE), NACC // NTILE)])
        plsc.subcore_barrier()

        def body(j, carry):
            pltpu.sync_copy(ones_v, deg_sh.at[cidx_v.at[j]], add=True)
            return carry

        lax.fori_loop(0, NWHIST, body, 0)
        plsc.subcore_barrier()
        sl = pl.ds(s * (NACC // NTILE), NACC // NTILE)
        pltpu.sync_copy(deg_sh.at[sl], out_hbm.at[c, sl])

    return hist(col_h, ones_w, zeros_row)


def _edge_pipeline(g_hbm, rows_hbm, cols_hbm, widx, nchunks,
                   ridx_v, cidx_v, bufs, gsems, ssems, acc_sh):
    def chunk(t, carry):
        pltpu.sync_copy(rows_hbm.at[widx, pl.ds(t * CHW, CHW)], ridx_v)
        pltpu.sync_copy(cols_hbm.at[widx, pl.ds(t * CHW, CHW)], cidx_v)
        descs_g = [None, None]
        descs_s = [None, None]
        descs_g[0] = pltpu.async_copy(g_hbm.at[ridx_v.at[0]], bufs[0], gsems[0])
        for j in range(CHW):
            b = j & 1
            nb = 1 - b
            if j >= 1:
                descs_s[nb].wait()
            if j + 1 < CHW:
                descs_g[nb] = pltpu.async_copy(
                    g_hbm.at[ridx_v.at[j + 1]], bufs[nb], gsems[nb])
            descs_g[b].wait()
            descs_s[b] = pltpu.async_copy(
                bufs[b], acc_sh.at[cidx_v.at[j]], ssems[b], add=True)
        descs_s[(CHW - 1) & 1].wait()
        return carry

    lax.fori_loop(0, nchunks, chunk, 0)


def _spmm(g0, g1, rows3, cols3, zeros_acc, F):
    @functools.partial(
        pl.kernel,
        out_type=(
            jax.ShapeDtypeStruct((NACC, F), jnp.float32),
            jax.ShapeDtypeStruct((NACC, F), jnp.float32),
        ),
        mesh=_sc_mesh(),
        scratch_types=[
            pltpu.VMEM((CHW, WIN), jnp.int32),
            pltpu.VMEM((CHW, WIN), jnp.int32),
            pltpu.VMEM((WIN, F), jnp.float32),
            pltpu.VMEM((WIN, F), jnp.float32),
            pltpu.VMEM_SHARED((NACC, F), jnp.float32),
            pltpu.SemaphoreType.DMA,
            pltpu.SemaphoreType.DMA,
            pltpu.SemaphoreType.DMA,
            pltpu.SemaphoreType.DMA,
        ],
    )
    def spmm(g0_hbm, g1_hbm, rows_hbm, cols_hbm, zeros_hbm,
             out0_hbm, out1_hbm, ridx_v, cidx_v, buf0, buf1, acc_sh,
             semg0, semg1, sems0, sems1):
        c = lax.axis_index("c")
        s = lax.axis_index("s")
        pltpu.sync_copy(zeros_hbm, acc_sh.at[pl.ds(s * (NACC // NTILE), NACC // NTILE)])
        plsc.subcore_barrier()

        @pl.when(c == 0)
        def _():
            _edge_pipeline(g0_hbm, rows_hbm, cols_hbm, s, NWMAIN // CHW,
                           ridx_v, cidx_v, (buf0, buf1), (semg0, semg1),
                           (sems0, sems1), acc_sh)

        @pl.when(c == 1)
        def _():
            _edge_pipeline(g1_hbm, rows_hbm, cols_hbm, s, NWMAIN // CHW,
                           ridx_v, cidx_v, (buf0, buf1), (semg0, semg1),
                           (sems0, sems1), acc_sh)

        plsc.subcore_barrier()
        osl = pl.ds(s * (NACC // NTILE), NACC // NTILE)

        @pl.when(c == 0)
        def _():
            pltpu.sync_copy(acc_sh.at[osl], out0_hbm.at[osl])

        @pl.when(c == 1)
        def _():
            pltpu.sync_copy(acc_sh.at[osl], out1_hbm.at[osl])

    return spmm(g0, g1, rows3, cols3, zeros_acc)


def _spmm_es(ga, gb, rows_w, cols_w, zeros_acc):
    @functools.partial(
        pl.kernel,
        out_type=(
            jax.ShapeDtypeStruct((NACC, 128), jnp.float32),
            jax.ShapeDtypeStruct((NACC, 128), jnp.float32),
        ),
        mesh=_sc_mesh(),
        scratch_types=[
            pltpu.VMEM((CHW, WIN), jnp.int32),
            pltpu.VMEM((CHW, WIN), jnp.int32),
            pltpu.VMEM((WIN, 128), jnp.float32),
            pltpu.VMEM((WIN, 128), jnp.float32),
            pltpu.VMEM_SHARED((NACC, 128), jnp.float32),
            pltpu.SemaphoreType.DMA,
            pltpu.SemaphoreType.DMA,
            pltpu.SemaphoreType.DMA,
            pltpu.SemaphoreType.DMA,
        ],
    )
    def spmm(ga_hbm, gb_hbm, rows_hbm, cols_hbm, zeros_hbm,
             out0_hbm, out1_hbm, ridx_v, cidx_v, buf0, buf1, acc_sh,
             semg0, semg1, sems0, sems1):
        c = lax.axis_index("c")
        s = lax.axis_index("s")
        w = s * 2 + c
        pltpu.sync_copy(zeros_hbm, acc_sh.at[pl.ds(s * (NACC // NTILE), NACC // NTILE)])
        plsc.subcore_barrier()

        @pl.when(c == 0)
        def _():
            _edge_pipeline(ga_hbm, rows_hbm, cols_hbm, w, NWHIST // CHW,
                           ridx_v, cidx_v, (buf0, buf1), (semg0, semg1),
                           (sems0, sems1), acc_sh)

        @pl.when(c == 1)
        def _():
            _edge_pipeline(gb_hbm, rows_hbm, cols_hbm, w, NWHIST // CHW,
                           ridx_v, cidx_v, (buf0, buf1), (semg0, semg1),
                           (sems0, sems1), acc_sh)

        plsc.subcore_barrier()
        osl = pl.ds(s * (NACC // NTILE), NACC // NTILE)

        @pl.when(c == 0)
        def _():
            pltpu.sync_copy(acc_sh.at[osl], out0_hbm.at[osl])

        @pl.when(c == 1)
        def _():
            pltpu.sync_copy(acc_sh.at[osl], out1_hbm.at[osl])

    return spmm(ga, gb, rows_w, cols_w, zeros_acc)


def _tc1(x, W1, d0, d1):
    def body(x_r, w_r, d0_r, d1_r, g1a_r, g1b_r, dis_r):
        deg = d0_r[...] + d1_r[...] + 1.0
        dis = lax.rsqrt(deg)
        g = dis * jnp.dot(x_r[...], w_r[...], preferred_element_type=jnp.float32)
        g1a_r[...] = g[:, :128]
        g1b_r[...] = g[:, 128:]
        dis_r[...] = dis

    return pl.pallas_call(
        body,
        grid=(N // RB,),
        in_specs=[
            pl.BlockSpec((RB, 128), lambda i: (i, 0)),
            pl.BlockSpec((128, 256), lambda i: (0, 0)),
            pl.BlockSpec((RB, 1), lambda i: (i, 0)),
            pl.BlockSpec((RB, 1), lambda i: (i, 0)),
        ],
        out_specs=[
            pl.BlockSpec((RB, 128), lambda i: (i, 0)),
            pl.BlockSpec((RB, 128), lambda i: (i, 0)),
            pl.BlockSpec((RB, 1), lambda i: (i, 0)),
        ],
        out_shape=[
            jax.ShapeDtypeStruct((N, 128), jnp.float32),
            jax.ShapeDtypeStruct((N, 128), jnp.float32),
            jax.ShapeDtypeStruct((N, 1), jnp.float32),
        ],
    )(x, W1, d0, d1)


def _tc2(s1a, s1b, g1a, g1b, dis, b1, W2):
    def body(s1a_r, s1b_r, g1a_r, g1b_r, dis_r, b1_r, w2_r, g2a_r, g2b_r):
        d = dis_r[...]
        pre0 = d * (s1a_r[...] + g1a_r[...]) + b1_r[:, :128]
        pre1 = d * (s1b_r[...] + g1b_r[...]) + b1_r[:, 128:]
        h2 = jnp.concatenate([jnp.maximum(pre0, 0.0), jnp.maximum(pre1, 0.0)], axis=1)
        g2 = d * jnp.dot(h2, w2_r[...], preferred_element_type=jnp.float32)
        g2a_r[...] = g2
        g2b_r[...] = g2

    return pl.pallas_call(
        body,
        grid=(N // RB,),
        in_specs=[
            pl.BlockSpec((RB, 128), lambda i: (i, 0)),
            pl.BlockSpec((RB, 128), lambda i: (i, 0)),
            pl.BlockSpec((RB, 128), lambda i: (i, 0)),
            pl.BlockSpec((RB, 128), lambda i: (i, 0)),
            pl.BlockSpec((RB, 1), lambda i: (i, 0)),
            pl.BlockSpec((1, 256), lambda i: (0, 0)),
            pl.BlockSpec((256, 128), lambda i: (0, 0)),
        ],
        out_specs=[
            pl.BlockSpec((RB, 128), lambda i: (i, 0)),
            pl.BlockSpec((RB, 128), lambda i: (i, 0)),
        ],
        out_shape=[
            jax.ShapeDtypeStruct((N, 128), jnp.float32),
            jax.ShapeDtypeStruct((N, 128), jnp.float32),
        ],
    )(s1a, s1b, g1a, g1b, dis, b1, W2)


def _tc3(s2p0, s2p1, g2, dis, b2):
    def body(p0_r, p1_r, g2_r, dis_r, b2_r, out_r):
        d = dis_r[...]
        out_r[...] = d * (p0_r[...] + p1_r[...] + g2_r[...]) + b2_r[...]

    return pl.pallas_call(
        body,
        grid=(N // RB,),
        in_specs=[
            pl.BlockSpec((RB, 128), lambda i: (i, 0)),
            pl.BlockSpec((RB, 128), lambda i: (i, 0)),
            pl.BlockSpec((RB, 128), lambda i: (i, 0)),
            pl.BlockSpec((RB, 1), lambda i: (i, 0)),
            pl.BlockSpec((1, 128), lambda i: (0, 0)),
        ],
        out_specs=pl.BlockSpec((RB, 128), lambda i: (i, 0)),
        out_shape=jax.ShapeDtypeStruct((N, 128), jnp.float32),
    )(s2p0, s2p1, g2, dis, b2)


def kernel(x, edge_index, W1, b1, W2, b2):
    row = edge_index[0].astype(jnp.int32)
    col = edge_index[1].astype(jnp.int32)

    pad_m = NTILE * NWMAIN * WIN - E
    junk_m = N + (jnp.arange(pad_m, dtype=jnp.int32) % (NACC - N))
    rows3 = jnp.concatenate([row, jnp.zeros((pad_m,), jnp.int32)]).reshape(NTILE, NWMAIN, WIN)
    cols3 = jnp.concatenate([col, junk_m]).reshape(NTILE, NWMAIN, WIN)

    pad_w = 32 * NWHIST * WIN - E
    junk_w = N + (jnp.arange(pad_w, dtype=jnp.int32) % (NACC - N))
    rows_w = jnp.concatenate([row, jnp.zeros((pad_w,), jnp.int32)]).reshape(32, NWHIST, WIN)
    cols_w = jnp.concatenate([col, junk_w]).reshape(32, NWHIST, WIN)

    ones_w = jnp.ones((WIN,), jnp.float32)
    zeros_row = jnp.zeros((NACC // NTILE,), jnp.float32)
    zeros_acc128 = jnp.zeros((NACC // NTILE, 128), jnp.float32)

    degp = _degree_hist(cols_w, ones_w, zeros_row)
    d0 = degp[0, :N].reshape(N, 1)
    d1 = degp[1, :N].reshape(N, 1)

    g1a, g1b, dis = _tc1(x, W1, d0, d1)
    s1a, s1b = _spmm(g1a, g1b, rows3, cols3, zeros_acc128, 128)
    g2a, g2b = _tc2(s1a, s1b, g1a, g1b, dis, b1.reshape(1, 256), W2)
    s2p0, s2p1 = _spmm_es(g2a, g2b, rows_w, cols_w, zeros_acc128)
    return _tc3(s2p0, s2p1, g2a, dis, b2.reshape(1, 128))


# speedup vs baseline: 26.0338x; 2.7118x over previous
import functools

import jax
import jax.numpy as jnp
from jax import lax
from jax.experimental import pallas as pl
from jax.experimental.pallas import tpu as pltpu
from jax.experimental.pallas import tpu_sc as plsc

N = 10000
E = 320000
NACC = 10240
WIN = 128
NTILE = 16
NWMAIN = 160
CHW = 16
NWHIST = 80
RB = 2000


def _sc_mesh():
    return plsc.VectorSubcoreMesh(core_axis_name="c", subcore_axis_name="s")


def _degree_hist(col_h, ones_w, zeros_row):
    @functools.partial(
        pl.kernel,
        out_type=jax.ShapeDtypeStruct((2, NACC), jnp.float32),
        mesh=_sc_mesh(),
        scratch_types=[
            pltpu.VMEM((NWHIST, WIN), jnp.int32),
            pltpu.VMEM((WIN,), jnp.float32),
            pltpu.VMEM_SHARED((NACC,), jnp.float32),
        ],
    )
    def hist(col_hbm, ones_hbm, zeros_hbm, out_hbm, cidx_v, ones_v, deg_sh):
        c = lax.axis_index("c")
        s = lax.axis_index("s")
        w = s * 2 + c
        pltpu.sync_copy(col_hbm.at[w], cidx_v)
        pltpu.sync_copy(ones_hbm, ones_v)
        pltpu.sync_copy(zeros_hbm, deg_sh.at[pl.ds(s * (NACC // NTILE), NACC // NTILE)])
        plsc.subcore_barrier()

        def body(j, carry):
            pltpu.sync_copy(ones_v, deg_sh.at[cidx_v.at[j]], add=True)
            return carry

        lax.fori_loop(0, NWHIST, body, 0)
        plsc.subcore_barrier()
        sl = pl.ds(s * (NACC // NTILE), NACC // NTILE)
        pltpu.sync_copy(deg_sh.at[sl], out_hbm.at[c, sl])

    return hist(col_h, ones_w, zeros_row)


def _edge_pipeline(g_hbm, rows_hbm, cols_hbm, widx, nchunks,
                   ridx_v, cidx_v, bufs, gsems, ssems, acc_sh):
    def chunk(t, carry):
        pltpu.sync_copy(rows_hbm.at[widx, pl.ds(t * CHW, CHW)], ridx_v)
        pltpu.sync_copy(cols_hbm.at[widx, pl.ds(t * CHW, CHW)], cidx_v)
        descs_g = [None, None]
        descs_s = [None, None]
        descs_g[0] = pltpu.async_copy(g_hbm.at[ridx_v.at[0]], bufs[0], gsems[0])
        for j in range(CHW):
            b = j & 1
            nb = 1 - b
            if j >= 1:
                descs_s[nb].wait()
            if j + 1 < CHW:
                descs_g[nb] = pltpu.async_copy(
                    g_hbm.at[ridx_v.at[j + 1]], bufs[nb], gsems[nb])
            descs_g[b].wait()
            descs_s[b] = pltpu.async_copy(
                bufs[b], acc_sh.at[cidx_v.at[j]], ssems[b], add=True)
        descs_s[(CHW - 1) & 1].wait()
        return carry

    lax.fori_loop(0, nchunks, chunk, 0)


def _spmm(g0, g1, rows3, cols3, zeros_acc, F):
    @functools.partial(
        pl.kernel,
        out_type=(
            jax.ShapeDtypeStruct((NACC, F), jnp.float32),
            jax.ShapeDtypeStruct((NACC, F), jnp.float32),
        ),
        mesh=_sc_mesh(),
        scratch_types=[
            pltpu.VMEM((CHW, WIN), jnp.int32),
            pltpu.VMEM((CHW, WIN), jnp.int32),
            pltpu.VMEM((WIN, F), jnp.float32),
            pltpu.VMEM((WIN, F), jnp.float32),
            pltpu.VMEM_SHARED((NACC, F), jnp.float32),
            pltpu.SemaphoreType.DMA,
            pltpu.SemaphoreType.DMA,
            pltpu.SemaphoreType.DMA,
            pltpu.SemaphoreType.DMA,
        ],
    )
    def spmm(g0_hbm, g1_hbm, rows_hbm, cols_hbm, zeros_hbm,
             out0_hbm, out1_hbm, ridx_v, cidx_v, buf0, buf1, acc_sh,
             semg0, semg1, sems0, sems1):
        c = lax.axis_index("c")
        s = lax.axis_index("s")
        pltpu.sync_copy(zeros_hbm, acc_sh.at[pl.ds(s * (NACC // NTILE), NACC // NTILE)])
        plsc.subcore_barrier()

        @pl.when(c == 0)
        def _():
            _edge_pipeline(g0_hbm, rows_hbm, cols_hbm, s, NWMAIN // CHW,
                           ridx_v, cidx_v, (buf0, buf1), (semg0, semg1),
                           (sems0, sems1), acc_sh)

        @pl.when(c == 1)
        def _():
            _edge_pipeline(g1_hbm, rows_hbm, cols_hbm, s, NWMAIN // CHW,
                           ridx_v, cidx_v, (buf0, buf1), (semg0, semg1),
                           (sems0, sems1), acc_sh)

        plsc.subcore_barrier()
        osl = pl.ds(s * (NACC // NTILE), NACC // NTILE)

        @pl.when(c == 0)
        def _():
            pltpu.sync_copy(acc_sh.at[osl], out0_hbm.at[osl])

        @pl.when(c == 1)
        def _():
            pltpu.sync_copy(acc_sh.at[osl], out1_hbm.at[osl])

    return spmm(g0, g1, rows3, cols3, zeros_acc)


def _spmm_es(ga, gb, rows_w, cols_w, zeros_acc):
    @functools.partial(
        pl.kernel,
        out_type=(
            jax.ShapeDtypeStruct((NACC, 128), jnp.float32),
            jax.ShapeDtypeStruct((NACC, 128), jnp.float32),
        ),
        mesh=_sc_mesh(),
        scratch_types=[
            pltpu.VMEM((CHW, WIN), jnp.int32),
            pltpu.VMEM((CHW, WIN), jnp.int32),
            pltpu.VMEM((WIN, 128), jnp.float32),
            pltpu.VMEM((WIN, 128), jnp.float32),
            pltpu.VMEM_SHARED((NACC, 128), jnp.float32),
            pltpu.SemaphoreType.DMA,
            pltpu.SemaphoreType.DMA,
            pltpu.SemaphoreType.DMA,
            pltpu.SemaphoreType.DMA,
        ],
    )
    def spmm(ga_hbm, gb_hbm, rows_hbm, cols_hbm, zeros_hbm,
             out0_hbm, out1_hbm, ridx_v, cidx_v, buf0, buf1, acc_sh,
             semg0, semg1, sems0, sems1):
        c = lax.axis_index("c")
        s = lax.axis_index("s")
        w = s * 2 + c
        pltpu.sync_copy(zeros_hbm, acc_sh.at[pl.ds(s * (NACC // NTILE), NACC // NTILE)])
        plsc.subcore_barrier()

        @pl.when(c == 0)
        def _():
            _edge_pipeline(ga_hbm, rows_hbm, cols_hbm, w, NWHIST // CHW,
                           ridx_v, cidx_v, (buf0, buf1), (semg0, semg1),
                           (sems0, sems1), acc_sh)

        @pl.when(c == 1)
        def _():
            _edge_pipeline(gb_hbm, rows_hbm, cols_hbm, w, NWHIST // CHW,
                           ridx_v, cidx_v, (buf0, buf1), (semg0, semg1),
                           (sems0, sems1), acc_sh)

        plsc.subcore_barrier()
        osl = pl.ds(s * (NACC // NTILE), NACC // NTILE)

        @pl.when(c == 0)
        def _():
            pltpu.sync_copy(acc_sh.at[osl], out0_hbm.at[osl])

        @pl.when(c == 1)
        def _():
            pltpu.sync_copy(acc_sh.at[osl], out1_hbm.at[osl])

    return spmm(ga, gb, rows_w, cols_w, zeros_acc)


def _tc1(x, W1, d0, d1):
    def body(x_r, w_r, d0_r, d1_r, g1a_r, g1b_r, dis_r):
        deg = d0_r[...] + d1_r[...] + 1.0
        dis = lax.rsqrt(deg)
        g = dis * jnp.dot(x_r[...], w_r[...], preferred_element_type=jnp.float32)
        g1a_r[...] = g[:, :128]
        g1b_r[...] = g[:, 128:]
        dis_r[...] = dis

    return pl.pallas_call(
        body,
        grid=(N // RB,),
        in_specs=[
            pl.BlockSpec((RB, 128), lambda i: (i, 0)),
            pl.BlockSpec((128, 256), lambda i: (0, 0)),
            pl.BlockSpec((RB, 1), lambda i: (i, 0)),
            pl.BlockSpec((RB, 1), lambda i: (i, 0)),
        ],
        out_specs=[
            pl.BlockSpec((RB, 128), lambda i: (i, 0)),
            pl.BlockSpec((RB, 128), lambda i: (i, 0)),
            pl.BlockSpec((RB, 1), lambda i: (i, 0)),
        ],
        out_shape=[
            jax.ShapeDtypeStruct((N, 128), jnp.float32),
            jax.ShapeDtypeStruct((N, 128), jnp.float32),
            jax.ShapeDtypeStruct((N, 1), jnp.float32),
        ],
    )(x, W1, d0, d1)


def _tc2(s1a, s1b, g1a, g1b, dis, b1, W2):
    def body(s1a_r, s1b_r, g1a_r, g1b_r, dis_r, b1_r, w2_r, g2a_r, g2b_r):
        d = dis_r[...]
        pre0 = d * (s1a_r[...] + g1a_r[...]) + b1_r[:, :128]
        pre1 = d * (s1b_r[...] + g1b_r[...]) + b1_r[:, 128:]
        h2 = jnp.concatenate([jnp.maximum(pre0, 0.0), jnp.maximum(pre1, 0.0)], axis=1)
        g2 = d * jnp.dot(h2, w2_r[...], preferred_element_type=jnp.float32)
        g2a_r[...] = g2
        g2b_r[...] = g2

    return pl.pallas_call(
        body,
        grid=(N // RB,),
        in_specs=[
            pl.BlockSpec((RB, 128), lambda i: (i, 0)),
            pl.BlockSpec((RB, 128), lambda i: (i, 0)),
            pl.BlockSpec((RB, 128), lambda i: (i, 0)),
            pl.BlockSpec((RB, 128), lambda i: (i, 0)),
            pl.BlockSpec((RB, 1), lambda i: (i, 0)),
            pl.BlockSpec((1, 256), lambda i: (0, 0)),
            pl.BlockSpec((256, 128), lambda i: (0, 0)),
        ],
        out_specs=[
            pl.BlockSpec((RB, 128), lambda i: (i, 0)),
            pl.BlockSpec((RB, 128), lambda i: (i, 0)),
        ],
        out_shape=[
            jax.ShapeDtypeStruct((N, 128), jnp.float32),
            jax.ShapeDtypeStruct((N, 128), jnp.float32),
        ],
    )(s1a, s1b, g1a, g1b, dis, b1, W2)


def _tc3(s2p0, s2p1, g2, dis, b2):
    def body(p0_r, p1_r, g2_r, dis_r, b2_r, out_r):
        d = dis_r[...]
        out_r[...] = d * (p0_r[...] + p1_r[...] + g2_r[...]) + b2_r[...]

    return pl.pallas_call(
        body,
        grid=(N // RB,),
        in_specs=[
            pl.BlockSpec((RB, 128), lambda i: (i, 0)),
            pl.BlockSpec((RB, 128), lambda i: (i, 0)),
            pl.BlockSpec((RB, 128), lambda i: (i, 0)),
            pl.BlockSpec((RB, 1), lambda i: (i, 0)),
            pl.BlockSpec((1, 128), lambda i: (0, 0)),
        ],
        out_specs=pl.BlockSpec((RB, 128), lambda i: (i, 0)),
        out_shape=jax.ShapeDtypeStruct((N, 128), jnp.float32),
    )(s2p0, s2p1, g2, dis, b2)


def kernel(x, edge_index, W1, b1, W2, b2):
    row = edge_index[0].astype(jnp.int32)
    col = edge_index[1].astype(jnp.int32)

    pad_m = NTILE * NWMAIN * WIN - E
    padrow_m = jnp.arange(pad_m, dtype=jnp.int32) % N
    junk_m = N + (jnp.arange(pad_m, dtype=jnp.int32) % (NACC - N))
    rows3 = jnp.concatenate([row, padrow_m]).reshape(NTILE, NWMAIN, WIN)
    cols3 = jnp.concatenate([col, junk_m]).reshape(NTILE, NWMAIN, WIN)

    pad_w = 32 * NWHIST * WIN - E
    padrow_w = jnp.arange(pad_w, dtype=jnp.int32) % N
    junk_w = N + (jnp.arange(pad_w, dtype=jnp.int32) % (NACC - N))
    rows_w = jnp.concatenate([row, padrow_w]).reshape(32, NWHIST, WIN)
    cols_w = jnp.concatenate([col, junk_w]).reshape(32, NWHIST, WIN)

    ones_w = jnp.ones((WIN,), jnp.float32)
    zeros_row = jnp.zeros((NACC // NTILE,), jnp.float32)
    zeros_acc128 = jnp.zeros((NACC // NTILE, 128), jnp.float32)

    degp = _degree_hist(cols_w, ones_w, zeros_row)
    d0 = degp[0, :N].reshape(N, 1)
    d1 = degp[1, :N].reshape(N, 1)

    g1a, g1b, dis = _tc1(x, W1, d0, d1)
    s1a, s1b = _spmm(g1a, g1b, rows3, cols3, zeros_acc128, 128)
    g2a, g2b = _tc2(s1a, s1b, g1a, g1b, dis, b1.reshape(1, 256), W2)
    s2p0, s2p1 = _spmm_es(g2a, g2b, rows_w, cols_w, zeros_acc128)
    return _tc3(s2p0, s2p1, g2a, dis, b2.reshape(1, 128))


# speedup vs baseline: 34.2159x; 1.3143x over previous
import functools

import jax
import jax.numpy as jnp
from jax import lax
from jax.experimental import pallas as pl
from jax.experimental.pallas import tpu as pltpu
from jax.experimental.pallas import tpu_sc as plsc

N = 10000
E = 320000
NACC = 10240
WIN = 128
NTILE = 16
CHW = 16
NWHIST = 80
RB = 2000


def _sc_mesh():
    return plsc.VectorSubcoreMesh(core_axis_name="c", subcore_axis_name="s")


def _degree_hist(col_h, ones_w, zeros_row):
    @functools.partial(
        pl.kernel,
        out_type=jax.ShapeDtypeStruct((2, NACC), jnp.float32),
        mesh=_sc_mesh(),
        scratch_types=[
            pltpu.VMEM((NWHIST, WIN), jnp.int32),
            pltpu.VMEM((WIN,), jnp.float32),
            pltpu.VMEM_SHARED((NACC,), jnp.float32),
        ],
    )
    def hist(col_hbm, ones_hbm, zeros_hbm, out_hbm, cidx_v, ones_v, deg_sh):
        c = lax.axis_index("c")
        s = lax.axis_index("s")
        w = s * 2 + c
        pltpu.sync_copy(col_hbm.at[w], cidx_v)
        pltpu.sync_copy(ones_hbm, ones_v)
        pltpu.sync_copy(zeros_hbm, deg_sh.at[pl.ds(s * (NACC // NTILE), NACC // NTILE)])
        plsc.subcore_barrier()

        def body(j, carry):
            pltpu.sync_copy(ones_v, deg_sh.at[cidx_v.at[j]], add=True)
            return carry

        lax.fori_loop(0, NWHIST, body, 0)
        plsc.subcore_barrier()
        sl = pl.ds(s * (NACC // NTILE), NACC // NTILE)
        pltpu.sync_copy(deg_sh.at[sl], out_hbm.at[c, sl])

    return hist(col_h, ones_w, zeros_row)


def _edge_pipeline(g_hbm, rows_hbm, cols_hbm, widx, nchunks,
                   ridx_v, cidx_v, bufs, gsems, ssems, acc_sh):
    def chunk(t, carry):
        pltpu.sync_copy(rows_hbm.at[widx, pl.ds(t * CHW, CHW)], ridx_v)
        pltpu.sync_copy(cols_hbm.at[widx, pl.ds(t * CHW, CHW)], cidx_v)
        descs_g = [None, None]
        descs_s = [None, None]
        descs_g[0] = pltpu.async_copy(g_hbm.at[ridx_v.at[0]], bufs[0], gsems[0])
        for j in range(CHW):
            b = j & 1
            nb = 1 - b
            if j >= 1:
                descs_s[nb].wait()
            if j + 1 < CHW:
                descs_g[nb] = pltpu.async_copy(
                    g_hbm.at[ridx_v.at[j + 1]], bufs[nb], gsems[nb])
            descs_g[b].wait()
            descs_s[b] = pltpu.async_copy(
                bufs[b], acc_sh.at[cidx_v.at[j]], ssems[b], add=True)
        descs_s[(CHW - 1) & 1].wait()
        return carry

    lax.fori_loop(0, nchunks, chunk, 0)


def _spmm_es(ga, gb, rows_w, cols_w, zeros_acc):
    @functools.partial(
        pl.kernel,
        out_type=(
            jax.ShapeDtypeStruct((NACC, 128), jnp.float32),
            jax.ShapeDtypeStruct((NACC, 128), jnp.float32),
        ),
        mesh=_sc_mesh(),
        scratch_types=[
            pltpu.VMEM((CHW, WIN), jnp.int32),
            pltpu.VMEM((CHW, WIN), jnp.int32),
            pltpu.VMEM((WIN, 128), jnp.float32),
            pltpu.VMEM((WIN, 128), jnp.float32),
            pltpu.VMEM_SHARED((NACC, 128), jnp.float32),
            pltpu.SemaphoreType.DMA,
            pltpu.SemaphoreType.DMA,
            pltpu.SemaphoreType.DMA,
            pltpu.SemaphoreType.DMA,
        ],
    )
    def spmm(ga_hbm, gb_hbm, rows_hbm, cols_hbm, zeros_hbm,
             out0_hbm, out1_hbm, ridx_v, cidx_v, buf0, buf1, acc_sh,
             semg0, semg1, sems0, sems1):
        c = lax.axis_index("c")
        s = lax.axis_index("s")
        w = s * 2 + c
        pltpu.sync_copy(zeros_hbm, acc_sh.at[pl.ds(s * (NACC // NTILE), NACC // NTILE)])
        plsc.subcore_barrier()

        @pl.when(c == 0)
        def _():
            _edge_pipeline(ga_hbm, rows_hbm, cols_hbm, w, NWHIST // CHW,
                           ridx_v, cidx_v, (buf0, buf1), (semg0, semg1),
                           (sems0, sems1), acc_sh)

        @pl.when(c == 1)
        def _():
            _edge_pipeline(gb_hbm, rows_hbm, cols_hbm, w, NWHIST // CHW,
                           ridx_v, cidx_v, (buf0, buf1), (semg0, semg1),
                           (sems0, sems1), acc_sh)

        plsc.subcore_barrier()
        osl = pl.ds(s * (NACC // NTILE), NACC // NTILE)

        @pl.when(c == 0)
        def _():
            pltpu.sync_copy(acc_sh.at[osl], out0_hbm.at[osl])

        @pl.when(c == 1)
        def _():
            pltpu.sync_copy(acc_sh.at[osl], out1_hbm.at[osl])

    return spmm(ga, gb, rows_w, cols_w, zeros_acc)


def _tcu(x, d0, d1):
    def body(x_r, d0_r, d1_r, ua_r, ub_r, dis_r):
        deg = d0_r[...] + d1_r[...] + 1.0
        dis = lax.rsqrt(deg)
        u = dis * x_r[...]
        ua_r[...] = u
        ub_r[...] = u
        dis_r[...] = dis

    return pl.pallas_call(
        body,
        grid=(N // RB,),
        in_specs=[
            pl.BlockSpec((RB, 128), lambda i: (i, 0)),
            pl.BlockSpec((RB, 1), lambda i: (i, 0)),
            pl.BlockSpec((RB, 1), lambda i: (i, 0)),
        ],
        out_specs=[
            pl.BlockSpec((RB, 128), lambda i: (i, 0)),
            pl.BlockSpec((RB, 128), lambda i: (i, 0)),
            pl.BlockSpec((RB, 1), lambda i: (i, 0)),
        ],
        out_shape=[
            jax.ShapeDtypeStruct((N, 128), jnp.float32),
            jax.ShapeDtypeStruct((N, 128), jnp.float32),
            jax.ShapeDtypeStruct((N, 1), jnp.float32),
        ],
    )(x, d0, d1)


def _tcmid(t0, t1, u, dis, b1, W1, W2):
    def body(t0_r, t1_r, u_r, dis_r, b1_r, w1_r, w2_r, g2a_r, g2b_r):
        d = dis_r[...]
        m = t0_r[...] + t1_r[...] + u_r[...]
        pre = d * jnp.dot(m, w1_r[...], preferred_element_type=jnp.float32) + b1_r[...]
        h2 = jnp.maximum(pre, 0.0)
        g2 = d * jnp.dot(h2, w2_r[...], preferred_element_type=jnp.float32)
        g2a_r[...] = g2
        g2b_r[...] = g2

    return pl.pallas_call(
        body,
        grid=(N // RB,),
        in_specs=[
            pl.BlockSpec((RB, 128), lambda i: (i, 0)),
            pl.BlockSpec((RB, 128), lambda i: (i, 0)),
            pl.BlockSpec((RB, 128), lambda i: (i, 0)),
            pl.BlockSpec((RB, 1), lambda i: (i, 0)),
            pl.BlockSpec((1, 256), lambda i: (0, 0)),
            pl.BlockSpec((128, 256), lambda i: (0, 0)),
            pl.BlockSpec((256, 128), lambda i: (0, 0)),
        ],
        out_specs=[
            pl.BlockSpec((RB, 128), lambda i: (i, 0)),
            pl.BlockSpec((RB, 128), lambda i: (i, 0)),
        ],
        out_shape=[
            jax.ShapeDtypeStruct((N, 128), jnp.float32),
            jax.ShapeDtypeStruct((N, 128), jnp.float32),
        ],
    )(t0, t1, u, dis, b1, W1, W2)


def _tc3(s2p0, s2p1, g2, dis, b2):
    def body(p0_r, p1_r, g2_r, dis_r, b2_r, out_r):
        d = dis_r[...]
        out_r[...] = d * (p0_r[...] + p1_r[...] + g2_r[...]) + b2_r[...]

    return pl.pallas_call(
        body,
        grid=(N // RB,),
        in_specs=[
            pl.BlockSpec((RB, 128), lambda i: (i, 0)),
            pl.BlockSpec((RB, 128), lambda i: (i, 0)),
            pl.BlockSpec((RB, 128), lambda i: (i, 0)),
            pl.BlockSpec((RB, 1), lambda i: (i, 0)),
            pl.BlockSpec((1, 128), lambda i: (0, 0)),
        ],
        out_specs=pl.BlockSpec((RB, 128), lambda i: (i, 0)),
        out_shape=jax.ShapeDtypeStruct((N, 128), jnp.float32),
    )(s2p0, s2p1, g2, dis, b2)


def kernel(x, edge_index, W1, b1, W2, b2):
    row = edge_index[0].astype(jnp.int32)
    col = edge_index[1].astype(jnp.int32)

    pad_w = 32 * NWHIST * WIN - E
    padrow_w = jnp.arange(pad_w, dtype=jnp.int32) % N
    junk_w = N + (jnp.arange(pad_w, dtype=jnp.int32) % (NACC - N))
    rows_w = jnp.concatenate([row, padrow_w]).reshape(32, NWHIST, WIN)
    cols_w = jnp.concatenate([col, junk_w]).reshape(32, NWHIST, WIN)

    ones_w = jnp.ones((WIN,), jnp.float32)
    zeros_row = jnp.zeros((NACC // NTILE,), jnp.float32)
    zeros_acc128 = jnp.zeros((NACC // NTILE, 128), jnp.float32)

    degp = _degree_hist(cols_w, ones_w, zeros_row)
    d0 = degp[0, :N].reshape(N, 1)
    d1 = degp[1, :N].reshape(N, 1)

    ua, ub, dis = _tcu(x, d0, d1)
    t0, t1 = _spmm_es(ua, ub, rows_w, cols_w, zeros_acc128)
    g2a, g2b = _tcmid(t0, t1, ua, dis, b1.reshape(1, 256), W1, W2)
    s2p0, s2p1 = _spmm_es(g2a, g2b, rows_w, cols_w, zeros_acc128)
    return _tc3(s2p0, s2p1, g2a, dis, b2.reshape(1, 128))


# speedup vs baseline: 34.2839x; 1.0020x over previous
import functools

import jax
import jax.numpy as jnp
from jax import lax
from jax.experimental import pallas as pl
from jax.experimental.pallas import tpu as pltpu
from jax.experimental.pallas import tpu_sc as plsc

N = 10000
E = 320000
NACC = 10240
WIN = 128
NTILE = 16
CHW = 16
NWHIST = 80
RB = 2000


def _sc_mesh():
    return plsc.VectorSubcoreMesh(core_axis_name="c", subcore_axis_name="s")


def _degree_hist(col_h, ones_w, zeros_row):
    @functools.partial(
        pl.kernel,
        out_type=jax.ShapeDtypeStruct((2, NACC), jnp.float32),
        mesh=_sc_mesh(),
        scratch_types=[
            pltpu.VMEM((NWHIST, WIN), jnp.int32),
            pltpu.VMEM((WIN,), jnp.float32),
            pltpu.VMEM_SHARED((NACC,), jnp.float32),
        ],
    )
    def hist(col_hbm, ones_hbm, zeros_hbm, out_hbm, cidx_v, ones_v, deg_sh):
        c = lax.axis_index("c")
        s = lax.axis_index("s")
        w = s * 2 + c
        pltpu.sync_copy(col_hbm.at[w], cidx_v)
        pltpu.sync_copy(ones_hbm, ones_v)
        pltpu.sync_copy(zeros_hbm, deg_sh.at[pl.ds(s * (NACC // NTILE), NACC // NTILE)])
        plsc.subcore_barrier()

        def body(j, carry):
            pltpu.sync_copy(ones_v, deg_sh.at[cidx_v.at[j]], add=True)
            return carry

        lax.fori_loop(0, NWHIST, body, 0)
        plsc.subcore_barrier()
        sl = pl.ds(s * (NACC // NTILE), NACC // NTILE)
        pltpu.sync_copy(deg_sh.at[sl], out_hbm.at[c, sl])

    return hist(col_h, ones_w, zeros_row)


def _edge_pipeline(g_hbm, rows_hbm, cols_hbm, widx, nchunks,
                   ridx_v, cidx_v, bufs, gsems, ssems, acc_sh):
    def chunk(t, carry):
        pltpu.sync_copy(rows_hbm.at[widx, pl.ds(t * CHW, CHW)], ridx_v)
        pltpu.sync_copy(cols_hbm.at[widx, pl.ds(t * CHW, CHW)], cidx_v)
        descs_g = [None, None]
        descs_s = [None, None]
        descs_g[0] = pltpu.async_copy(g_hbm.at[ridx_v.at[0]], bufs[0], gsems[0])
        for j in range(CHW):
            b = j & 1
            nb = 1 - b
            if j >= 1:
                descs_s[nb].wait()
            if j + 1 < CHW:
                descs_g[nb] = pltpu.async_copy(
                    g_hbm.at[ridx_v.at[j + 1]], bufs[nb], gsems[nb])
            descs_g[b].wait()
            descs_s[b] = pltpu.async_copy(
                bufs[b], acc_sh.at[cidx_v.at[j]], ssems[b], add=True)
        descs_s[(CHW - 1) & 1].wait()
        return carry

    lax.fori_loop(0, nchunks, chunk, 0)


def _spmm_es(ga, gb, rows_w, cols_w, zeros_acc):
    @functools.partial(
        pl.kernel,
        out_type=(
            jax.ShapeDtypeStruct((NACC, 128), jnp.float32),
            jax.ShapeDtypeStruct((NACC, 128), jnp.float32),
        ),
        mesh=_sc_mesh(),
        scratch_types=[
            pltpu.VMEM((CHW, WIN), jnp.int32),
            pltpu.VMEM((CHW, WIN), jnp.int32),
            pltpu.VMEM((WIN, 128), jnp.float32),
            pltpu.VMEM((WIN, 128), jnp.float32),
            pltpu.VMEM_SHARED((NACC, 128), jnp.float32),
            pltpu.SemaphoreType.DMA,
            pltpu.SemaphoreType.DMA,
            pltpu.SemaphoreType.DMA,
            pltpu.SemaphoreType.DMA,
        ],
    )
    def spmm(ga_hbm, gb_hbm, rows_hbm, cols_hbm, zeros_hbm,
             out0_hbm, out1_hbm, ridx_v, cidx_v, buf0, buf1, acc_sh,
             semg0, semg1, sems0, sems1):
        c = lax.axis_index("c")
        s = lax.axis_index("s")
        w = s * 2 + c
        pltpu.sync_copy(zeros_hbm, acc_sh.at[pl.ds(s * (NACC // NTILE), NACC // NTILE)])
        plsc.subcore_barrier()

        @pl.when(c == 0)
        def _():
            _edge_pipeline(ga_hbm, rows_hbm, cols_hbm, w, NWHIST // CHW,
                           ridx_v, cidx_v, (buf0, buf1), (semg0, semg1),
                           (sems0, sems1), acc_sh)

        @pl.when(c == 1)
        def _():
            _edge_pipeline(gb_hbm, rows_hbm, cols_hbm, w, NWHIST // CHW,
                           ridx_v, cidx_v, (buf0, buf1), (semg0, semg1),
                           (sems0, sems1), acc_sh)

        plsc.subcore_barrier()
        osl = pl.ds(s * (NACC // NTILE), NACC // NTILE)

        @pl.when(c == 0)
        def _():
            pltpu.sync_copy(acc_sh.at[osl], out0_hbm.at[osl])

        @pl.when(c == 1)
        def _():
            pltpu.sync_copy(acc_sh.at[osl], out1_hbm.at[osl])

    return spmm(ga, gb, rows_w, cols_w, zeros_acc)


def _tcu(x, d0, d1):
    def body(x_r, d0_r, d1_r, ua_r, ub_r, dis_r):
        deg = d0_r[...] + d1_r[...] + 1.0
        dis = lax.rsqrt(deg)
        u = dis * x_r[...]
        ua_r[...] = u
        ub_r[...] = u
        dis_r[...] = dis

    return pl.pallas_call(
        body,
        grid=(N // RB,),
        in_specs=[
            pl.BlockSpec((RB, 128), lambda i: (i, 0)),
            pl.BlockSpec((RB, 1), lambda i: (i, 0)),
            pl.BlockSpec((RB, 1), lambda i: (i, 0)),
        ],
        out_specs=[
            pl.BlockSpec((RB, 128), lambda i: (i, 0)),
            pl.BlockSpec((RB, 128), lambda i: (i, 0)),
            pl.BlockSpec((RB, 1), lambda i: (i, 0)),
        ],
        out_shape=[
            jax.ShapeDtypeStruct((N, 128), jnp.float32),
            jax.ShapeDtypeStruct((N, 128), jnp.float32),
            jax.ShapeDtypeStruct((N, 1), jnp.float32),
        ],
    )(x, d0, d1)


def _tcmid(t0, t1, u, dis, b1, W1, W2):
    def body(t0_r, t1_r, u_r, dis_r, b1_r, w1_r, w2_r, g2a_r, g2b_r):
        d = dis_r[...]
        m = t0_r[...] + t1_r[...] + u_r[...]
        pre = d * jnp.dot(m, w1_r[...], preferred_element_type=jnp.float32) + b1_r[...]
        h2 = jnp.maximum(pre, 0.0)
        g2 = d * jnp.dot(h2, w2_r[...], preferred_element_type=jnp.float32)
        g2a_r[...] = g2
        g2b_r[...] = g2

    return pl.pallas_call(
        body,
        grid=(N // RB,),
        in_specs=[
            pl.BlockSpec((RB, 128), lambda i: (i, 0)),
            pl.BlockSpec((RB, 128), lambda i: (i, 0)),
            pl.BlockSpec((RB, 128), lambda i: (i, 0)),
            pl.BlockSpec((RB, 1), lambda i: (i, 0)),
            pl.BlockSpec((1, 256), lambda i: (0, 0)),
            pl.BlockSpec((128, 256), lambda i: (0, 0)),
            pl.BlockSpec((256, 128), lambda i: (0, 0)),
        ],
        out_specs=[
            pl.BlockSpec((RB, 128), lambda i: (i, 0)),
            pl.BlockSpec((RB, 128), lambda i: (i, 0)),
        ],
        out_shape=[
            jax.ShapeDtypeStruct((N, 128), jnp.float32),
            jax.ShapeDtypeStruct((N, 128), jnp.float32),
        ],
    )(t0, t1, u, dis, b1, W1, W2)


def _tc3(s2p0, s2p1, g2, dis, b2):
    def body(p0_r, p1_r, g2_r, dis_r, b2_r, out_r):
        d = dis_r[...]
        out_r[...] = d * (p0_r[...] + p1_r[...] + g2_r[...]) + b2_r[...]

    return pl.pallas_call(
        body,
        grid=(N // RB,),
        in_specs=[
            pl.BlockSpec((RB, 128), lambda i: (i, 0)),
            pl.BlockSpec((RB, 128), lambda i: (i, 0)),
            pl.BlockSpec((RB, 128), lambda i: (i, 0)),
            pl.BlockSpec((RB, 1), lambda i: (i, 0)),
            pl.BlockSpec((1, 128), lambda i: (0, 0)),
        ],
        out_specs=pl.BlockSpec((RB, 128), lambda i: (i, 0)),
        out_shape=jax.ShapeDtypeStruct((N, 128), jnp.float32),
    )(s2p0, s2p1, g2, dis, b2)


def kernel(x, edge_index, W1, b1, W2, b2):
    row = edge_index[0].astype(jnp.int32)
    col = edge_index[1].astype(jnp.int32)

    pad_w = 32 * NWHIST * WIN - E
    padrow_w = jnp.arange(pad_w, dtype=jnp.int32) % N
    junk_w = N + (jnp.arange(pad_w, dtype=jnp.int32) % (NACC - N))
    rows_w = jnp.concatenate([row, padrow_w]).reshape(32, NWHIST, WIN)
    cols_w = jnp.concatenate([col, junk_w]).reshape(32, NWHIST, WIN)

    ones_w = jnp.ones((WIN,), jnp.float32)
    zeros_row = jnp.zeros((NACC // NTILE,), jnp.float32)
    zeros_acc128 = jnp.zeros((NACC // NTILE, 128), jnp.float32)

    degp = _degree_hist(cols_w, ones_w, zeros_row)
    d0 = degp[0, :N].reshape(N, 1)
    d1 = degp[1, :N].reshape(N, 1)
    rows_w = lax.optimization_barrier((rows_w, d0))[0]

    ua, ub, dis = _tcu(x, d0, d1)
    t0, t1 = _spmm_es(ua, ub, rows_w, cols_w, zeros_acc128)
    g2a, g2b = _tcmid(t0, t1, ua, dis, b1.reshape(1, 256), W1, W2)
    s2p0, s2p1 = _spmm_es(g2a, g2b, rows_w, cols_w, zeros_acc128)
    return _tc3(s2p0, s2p1, g2a, dis, b2.reshape(1, 128))
